# baseline (speedup 1.0000x reference)
import functools

import jax
import jax.numpy as jnp
from jax import lax
from jax.experimental import pallas as pl
from jax.experimental.pallas import tpu as pltpu
from jax.experimental.pallas import tpu_sc as plsc

F32 = jnp.float32
BF16 = jnp.bfloat16
B, C, T = 16, 256, 256
NB = 8
W = NB * T
NROWS = B * C
NWORK = 32
RPW = NROWS // NWORK


def _sc_gather(x_flat, gidx_flat):
    mesh = plsc.VectorSubcoreMesh(core_axis_name="c", subcore_axis_name="s")

    @functools.partial(
        pl.kernel, mesh=mesh,
        out_type=jax.ShapeDtypeStruct((NROWS, C), F32),
        scratch_types=[
            pltpu.VMEM((RPW,), jnp.int32),
            pltpu.VMEM((RPW, C), F32),
            pltpu.SemaphoreType.DMA,
        ],
    )
    def k(table_hbm, idx_hbm, out_hbm, idx_v, rows_v, sem):
        wid = lax.axis_index("s") * 2 + lax.axis_index("c")
        base = wid * RPW
        pltpu.sync_copy(idx_hbm.at[pl.ds(base, RPW)], idx_v)
        pltpu.async_copy(table_hbm.at[idx_v], rows_v, sem).wait()
        pltpu.sync_copy(rows_v, out_hbm.at[pl.ds(base, RPW)])

    return k(x_flat, gidx_flat)


def _tc_helpers(vecs):
    def vrow(i):
        return vecs[i:i + 1, :]

    def vtile(i):
        r = vrow(i)
        return jnp.concatenate([r] * NB, axis=1)

    def vcolT(i):
        return jnp.transpose(vrow(i))

    return vrow, vtile, vcolT


def _make_vp(w6, vecs):
    vrow, vtile, vcolT = _tc_helpers(vecs)

    colw = lax.broadcasted_iota(jnp.int32, (1, W), 1)
    tmod = jnp.bitwise_and(colw, T - 1)
    mask_first = (tmod != 0).astype(BF16)
    mask_last = (tmod != T - 1).astype(BF16)

    rW = lax.broadcasted_iota(jnp.int32, (W, NB), 0)
    cW = lax.broadcasted_iota(jnp.int32, (W, NB), 1)
    bd01 = ((rW // T) == cW).astype(F32)
    bd = bd01 * (1.0 / T)
    rWt = lax.broadcasted_iota(jnp.int32, (NB, W), 0)
    cWt = lax.broadcasted_iota(jnp.int32, (NB, W), 1)
    bdt = ((cWt // T) == rWt).astype(F32)

    def shifts(xb):
        xm = jnp.concatenate([jnp.zeros((C, 1), BF16), xb[:, :-1]],
                             axis=1) * mask_first
        xp = jnp.concatenate([xb[:, 1:], jnp.zeros((C, 1), BF16)],
                             axis=1) * mask_last
        return xm, xp

    def conv_big(xb, wi, bcol):
        w = w6[wi]
        xm, xp = shifts(xb)
        a = jnp.dot(w[0:C, :], xm, preferred_element_type=F32)
        a = a + jnp.dot(w[C:2 * C, :], xb, preferred_element_type=F32)
        a = a + jnp.dot(w[2 * C:3 * C, :], xp, preferred_element_type=F32)
        return a + bcol

    def ln_big(h, gbig, bebig):
        mu_s = jnp.dot(h, bd, preferred_element_type=F32)
        mu = jnp.dot(mu_s, bdt, preferred_element_type=F32)
        hc = h - mu
        var_s = jnp.dot(hc * hc, bd, preferred_element_type=F32)
        r = lax.rsqrt(var_s + 1e-5)
        rb = jnp.dot(r, bdt, preferred_element_type=F32)
        return hc * rb * gbig + bebig

    def vp_big(xb, wi, v0, lb):
        h = jnp.maximum(conv_big(xb, wi, vcolT(v0)), 0.0)
        h = ln_big(h, vtile(v0 + 1), vtile(v0 + 2))
        h2 = jnp.maximum(conv_big(h.astype(BF16), wi + 1, vcolT(v0 + 3)), 0.0)
        h2 = ln_big(h2, vtile(v0 + 4), vtile(v0 + 5))
        pred = jnp.dot(h2 * vtile(v0 + 6), bd01, preferred_element_type=F32)
        return jnp.transpose(pred + lb)

    return vp_big


def _tca_body(maxlen_ref, lbs_ref, x_ref, dur_ref, w6, vecs,
              dpred_ref, gidx_ref, vmask_ref, mel_ref):
    vp_big = _make_vp(w6, vecs)
    xbig = jnp.concatenate([x_ref[i].astype(BF16) for i in range(NB)], axis=1)
    dpred_ref[0] = vp_big(xbig, 0, 0, lbs_ref[0])

    col_i = lax.broadcasted_iota(jnp.int32, (C, C), 1).astype(F32)
    row_i = lax.broadcasted_iota(jnp.int32, (C, C), 0).astype(F32)
    upper = (row_i <= col_i).astype(BF16)
    prow = lax.broadcasted_iota(jnp.int32, (1, C), 1).astype(F32)
    ones_row = jnp.full((1, C), 1.0, BF16)
    maxlen_f = maxlen_ref[0].astype(F32)
    step = pl.program_id(0)

    dmat = dur_ref[...].astype(BF16)
    cs = jnp.dot(dmat, upper, preferred_element_type=F32)
    totals = cs[:, T - 1:T]
    for b in range(NB):
        cs_col = jnp.transpose(cs[b:b + 1, :])
        cmp = (col_i >= cs_col).astype(BF16)
        idx = jnp.dot(ones_row, cmp, preferred_element_type=F32)
        idx = jnp.minimum(idx, C - 1)
        total_b = totals[b:b + 1, 0:1]
        valid = (prow < total_b) & (prow < maxlen_f)
        boff = (step * NB + b) * C
        gidx_ref[b:b + 1, :] = idx.astype(jnp.int32) + boff
        vmask_ref[b:b + 1, :] = valid.astype(F32)
    mel_ref[...] = totals.astype(jnp.int32)


def _tcb_body(lbs_ref, x2_ref, vmask_ref, pt_ref, et_ref, w6, vecs,
              out_ref, ppred_ref, epred_ref):
    vp_big = _make_vp(w6, vecs)
    vrow, _, _ = _tc_helpers(vecs)

    vm = jnp.transpose(vmask_ref[...])
    parts2 = [x2_ref[i] * vm[:, i:i + 1] for i in range(NB)]
    x2 = jnp.concatenate(parts2, axis=1)
    x2b = x2.astype(BF16)

    ppred_ref[0] = vp_big(x2b, 2, 7, lbs_ref[1])
    epred_ref[0] = vp_big(x2b, 4, 14, lbs_ref[2])

    ptcols = jnp.transpose(pt_ref[...])
    etcols = jnp.transpose(et_ref[...])
    for b in range(NB):
        pc = ptcols[:, b:b + 1]
        ec = etcols[:, b:b + 1]
        pcm = jnp.concatenate([jnp.zeros((1, 1), F32), pc[:-1, :]], axis=0)
        pcp = jnp.concatenate([pc[1:, :], jnp.zeros((1, 1), F32)], axis=0)
        ecm = jnp.concatenate([jnp.zeros((1, 1), F32), ec[:-1, :]], axis=0)
        ecp = jnp.concatenate([ec[1:, :], jnp.zeros((1, 1), F32)], axis=0)
        emb = (pcm * vrow(21) + pc * vrow(22) + pcp * vrow(23) + vrow(27)
               + ecm * vrow(24) + ec * vrow(25) + ecp * vrow(26) + vrow(28))
        out_ref[b] = parts2[b] + emb


def _full(shape):
    nd = len(shape)
    return pl.BlockSpec(shape, lambda b: (0,) * nd)


def kernel(x, src_len, duration_target, pitch_target, energy_target, max_len,
           dp_w1, dp_b1, dp_g1, dp_be1, dp_w2, dp_b2, dp_g2, dp_be2, dp_lw, dp_lb,
           pp_w1, pp_b1, pp_g1, pp_be1, pp_w2, pp_b2, pp_g2, pp_be2, pp_lw, pp_lb,
           ep_w1, ep_b1, ep_g1, ep_be1, ep_w2, ep_b2, ep_g2, ep_be2, ep_lw, ep_lb,
           pe_w, pe_b, ee_w, ee_b):
    del src_len
    maxlen = jnp.asarray(max_len, jnp.int32).reshape(1)
    lbs = jnp.concatenate([dp_lb, pp_lb, ep_lb]).astype(F32)

    w6 = jnp.stack([
        jnp.transpose(w, (2, 0, 1)).reshape(3 * C, C)
        for w in (dp_w1, dp_w2, pp_w1, pp_w2, ep_w1, ep_w2)
    ]).astype(BF16)

    vecs = jnp.stack([
        dp_b1, dp_g1, dp_be1, dp_b2, dp_g2, dp_be2, dp_lw.reshape(T),
        pp_b1, pp_g1, pp_be1, pp_b2, pp_g2, pp_be2, pp_lw.reshape(T),
        ep_b1, ep_g1, ep_be1, ep_b2, ep_g2, ep_be2, ep_lw.reshape(T),
        pe_w[:, 0, 0], pe_w[:, 0, 1], pe_w[:, 0, 2],
        ee_w[:, 0, 0], ee_w[:, 0, 1], ee_w[:, 0, 2],
        pe_b, ee_b,
    ]).astype(F32)

    dur = duration_target.astype(jnp.int32)

    dpred, gidx, vmask, mel = pl.pallas_call(
        _tca_body,
        grid=(B // NB,),
        in_specs=[pl.BlockSpec(memory_space=pltpu.SMEM),
                  pl.BlockSpec(memory_space=pltpu.SMEM),
                  pl.BlockSpec((NB, C, T), lambda s: (s, 0, 0)),
                  pl.BlockSpec((NB, T), lambda s: (s, 0)),
                  _full((6, 3 * C, C)), _full((29, T))],
        out_specs=(pl.BlockSpec((1, NB, C), lambda s: (s, 0, 0)),
                   pl.BlockSpec((NB, T), lambda s: (s, 0)),
                   pl.BlockSpec((NB, T), lambda s: (s, 0)),
                   pl.BlockSpec((NB, 1), lambda s: (s, 0))),
        out_shape=(jax.ShapeDtypeStruct((B // NB, NB, C), F32),
                   jax.ShapeDtypeStruct((B, T), jnp.int32),
                   jax.ShapeDtypeStruct((B, T), F32),
                   jax.ShapeDtypeStruct((B, 1), jnp.int32)),
        compiler_params=pltpu.CompilerParams(
            dimension_semantics=("arbitrary",)),
    )(maxlen, lbs, x, dur, w6, vecs)

    x2_flat = _sc_gather(x.reshape(NROWS, C), gidx.reshape(NROWS))

    out, ppred, epred = pl.pallas_call(
        _tcb_body,
        grid=(B // NB,),
        in_specs=[pl.BlockSpec(memory_space=pltpu.SMEM),
                  pl.BlockSpec((NB, C, T), lambda s: (s, 0, 0)),
                  pl.BlockSpec((NB, T), lambda s: (s, 0)),
                  pl.BlockSpec((NB, T), lambda s: (s, 0)),
                  pl.BlockSpec((NB, T), lambda s: (s, 0)),
                  _full((6, 3 * C, C)), _full((29, T))],
        out_specs=(pl.BlockSpec((NB, C, T), lambda s: (s, 0, 0)),
                   pl.BlockSpec((1, NB, C), lambda s: (s, 0, 0)),
                   pl.BlockSpec((1, NB, C), lambda s: (s, 0, 0))),
        out_shape=(jax.ShapeDtypeStruct((B, C, T), F32),
                   jax.ShapeDtypeStruct((B // NB, NB, C), F32),
                   jax.ShapeDtypeStruct((B // NB, NB, C), F32)),
        compiler_params=pltpu.CompilerParams(
            dimension_semantics=("parallel",)),
    )(lbs, x2_flat.reshape(B, C, T), vmask, pitch_target, energy_target,
      w6, vecs)

    return (out, mel.reshape(B), dpred.reshape(B, C), ppred.reshape(B, C),
            epred.reshape(B, C))


# speedup vs baseline: 1.0918x; 1.0918x over previous
import functools

import jax
import jax.numpy as jnp
from jax import lax
from jax.experimental import pallas as pl
from jax.experimental.pallas import tpu as pltpu
from jax.experimental.pallas import tpu_sc as plsc

F32 = jnp.float32
BF16 = jnp.bfloat16
B, C, T = 16, 256, 256
NB = 8
W = NB * T
NROWS = B * C
NWORK = 32
RPW = NROWS // NWORK


def _tc0_body(maxlen_ref, dur_ref, gidx_ref, vmask_ref, mel_ref):
    row_i = lax.broadcasted_iota(jnp.int32, (C, C), 0).astype(F32)
    col_i = lax.broadcasted_iota(jnp.int32, (C, C), 1).astype(F32)
    upper = (row_i <= col_i).astype(BF16)
    prow = lax.broadcasted_iota(jnp.int32, (1, C), 1).astype(F32)
    ones_row = jnp.full((1, C), 1.0, BF16)
    maxlen_f = maxlen_ref[0].astype(F32)

    dmat = dur_ref[...].astype(BF16)
    cs = jnp.dot(dmat, upper, preferred_element_type=F32)
    totals = cs[:, T - 1:T]
    for b in range(B):
        cs_col = jnp.transpose(cs[b:b + 1, :])
        cmp = (col_i >= cs_col).astype(BF16)
        idx = jnp.dot(ones_row, cmp, preferred_element_type=F32)
        idx = jnp.minimum(idx, C - 1)
        total_b = totals[b:b + 1, 0:1]
        valid = (prow < total_b) & (prow < maxlen_f)
        gidx_ref[b:b + 1, :] = (idx + C * b).astype(jnp.int32)
        vmask_ref[b:b + 1, :] = valid.astype(F32)
    mel_ref[...] = totals.astype(jnp.int32)


def _tc0(maxlen, dur):
    return pl.pallas_call(
        _tc0_body,
        grid=(1,),
        in_specs=[pl.BlockSpec(memory_space=pltpu.SMEM),
                  pl.BlockSpec((B, T), lambda i: (0, 0))],
        out_specs=(pl.BlockSpec((B, T), lambda i: (0, 0)),
                   pl.BlockSpec((B, T), lambda i: (0, 0)),
                   pl.BlockSpec((B, 1), lambda i: (0, 0))),
        out_shape=(jax.ShapeDtypeStruct((B, T), jnp.int32),
                   jax.ShapeDtypeStruct((B, T), F32),
                   jax.ShapeDtypeStruct((B, 1), jnp.int32)),
    )(maxlen, dur)


def _sc_gather(x_flat, gidx_flat):
    mesh = plsc.VectorSubcoreMesh(core_axis_name="c", subcore_axis_name="s")

    @functools.partial(
        pl.kernel, mesh=mesh,
        out_type=jax.ShapeDtypeStruct((NROWS, C), F32),
        scratch_types=[
            pltpu.VMEM((RPW,), jnp.int32),
            pltpu.VMEM((RPW, C), F32),
            pltpu.SemaphoreType.DMA,
        ],
    )
    def k(table_hbm, idx_hbm, out_hbm, idx_v, rows_v, sem):
        wid = lax.axis_index("s") * 2 + lax.axis_index("c")
        base = wid * RPW
        pltpu.sync_copy(idx_hbm.at[pl.ds(base, RPW)], idx_v)
        pltpu.async_copy(table_hbm.at[idx_v], rows_v, sem).wait()
        pltpu.sync_copy(rows_v, out_hbm.at[pl.ds(base, RPW)])

    return k(x_flat, gidx_flat)


def _tc_helpers(vecs):
    def vrow(i):
        return vecs[i:i + 1, :]

    def vtile(i):
        r = vrow(i)
        return jnp.concatenate([r] * NB, axis=1)

    def vcolT(i):
        return jnp.transpose(vrow(i))

    return vrow, vtile, vcolT


def _make_vp(w6, vecs):
    vrow, vtile, vcolT = _tc_helpers(vecs)

    colw = lax.broadcasted_iota(jnp.int32, (1, W), 1)
    tmod = jnp.bitwise_and(colw, T - 1)
    mask_first = (tmod != 0).astype(BF16)
    mask_last = (tmod != T - 1).astype(BF16)

    rW = lax.broadcasted_iota(jnp.int32, (W, NB), 0)
    cW = lax.broadcasted_iota(jnp.int32, (W, NB), 1)
    bd01 = ((rW // T) == cW).astype(F32)
    bd = bd01 * (1.0 / T)
    rWt = lax.broadcasted_iota(jnp.int32, (NB, W), 0)
    cWt = lax.broadcasted_iota(jnp.int32, (NB, W), 1)
    bdt = ((cWt // T) == rWt).astype(F32)

    def shifts(xb):
        xm = jnp.concatenate([jnp.zeros((C, 1), BF16), xb[:, :-1]],
                             axis=1) * mask_first
        xp = jnp.concatenate([xb[:, 1:], jnp.zeros((C, 1), BF16)],
                             axis=1) * mask_last
        return xm, xp

    def conv_big(xb, wi, bcol):
        w = w6[wi]
        xm, xp = shifts(xb)
        a = jnp.dot(w[0:C, :], xm, preferred_element_type=F32)
        a = a + jnp.dot(w[C:2 * C, :], xb, preferred_element_type=F32)
        a = a + jnp.dot(w[2 * C:3 * C, :], xp, preferred_element_type=F32)
        return a + bcol

    def ln_big(h, gbig, bebig):
        mu_s = jnp.dot(h, bd, preferred_element_type=F32)
        mu = jnp.dot(mu_s, bdt, preferred_element_type=F32)
        hc = h - mu
        var_s = jnp.dot(hc * hc, bd, preferred_element_type=F32)
        r = lax.rsqrt(var_s + 1e-5)
        rb = jnp.dot(r, bdt, preferred_element_type=F32)
        return hc * rb * gbig + bebig

    def vp_big(xb, wi, v0, lb):
        h = jnp.maximum(conv_big(xb, wi, vcolT(v0)), 0.0)
        h = ln_big(h, vtile(v0 + 1), vtile(v0 + 2))
        h2 = jnp.maximum(conv_big(h.astype(BF16), wi + 1, vcolT(v0 + 3)), 0.0)
        h2 = ln_big(h2, vtile(v0 + 4), vtile(v0 + 5))
        pred = jnp.dot(h2 * vtile(v0 + 6), bd01, preferred_element_type=F32)
        return jnp.transpose(pred + lb)

    return vp_big


def _tca_body(lbs_ref, x_ref, w6, vecs, dpred_ref):
    vp_big = _make_vp(w6, vecs)
    xbig = jnp.concatenate([x_ref[i].astype(BF16) for i in range(NB)], axis=1)
    dpred_ref[0] = vp_big(xbig, 0, 0, lbs_ref[0])


def _tcb_body(lbs_ref, x2_ref, vmask_ref, pt_ref, et_ref, w6, vecs,
              out_ref, ppred_ref, epred_ref):
    vp_big = _make_vp(w6, vecs)
    vrow, _, _ = _tc_helpers(vecs)

    vm = jnp.transpose(vmask_ref[...])
    parts2 = [x2_ref[i] * vm[:, i:i + 1] for i in range(NB)]
    x2 = jnp.concatenate(parts2, axis=1)
    x2b = x2.astype(BF16)

    ppred_ref[0] = vp_big(x2b, 2, 7, lbs_ref[1])
    epred_ref[0] = vp_big(x2b, 4, 14, lbs_ref[2])

    ptcols = jnp.transpose(pt_ref[...])
    etcols = jnp.transpose(et_ref[...])
    for b in range(NB):
        pc = ptcols[:, b:b + 1]
        ec = etcols[:, b:b + 1]
        pcm = jnp.concatenate([jnp.zeros((1, 1), F32), pc[:-1, :]], axis=0)
        pcp = jnp.concatenate([pc[1:, :], jnp.zeros((1, 1), F32)], axis=0)
        ecm = jnp.concatenate([jnp.zeros((1, 1), F32), ec[:-1, :]], axis=0)
        ecp = jnp.concatenate([ec[1:, :], jnp.zeros((1, 1), F32)], axis=0)
        emb = (pcm * vrow(21) + pc * vrow(22) + pcp * vrow(23) + vrow(27)
               + ecm * vrow(24) + ec * vrow(25) + ecp * vrow(26) + vrow(28))
        out_ref[b] = parts2[b] + emb


def _full(shape):
    nd = len(shape)
    return pl.BlockSpec(shape, lambda b: (0,) * nd)


def kernel(x, src_len, duration_target, pitch_target, energy_target, max_len,
           dp_w1, dp_b1, dp_g1, dp_be1, dp_w2, dp_b2, dp_g2, dp_be2, dp_lw, dp_lb,
           pp_w1, pp_b1, pp_g1, pp_be1, pp_w2, pp_b2, pp_g2, pp_be2, pp_lw, pp_lb,
           ep_w1, ep_b1, ep_g1, ep_be1, ep_w2, ep_b2, ep_g2, ep_be2, ep_lw, ep_lb,
           pe_w, pe_b, ee_w, ee_b):
    del src_len
    maxlen = jnp.asarray(max_len, jnp.int32).reshape(1)
    lbs = jnp.concatenate([dp_lb, pp_lb, ep_lb]).astype(F32)

    w6 = jnp.stack([
        jnp.transpose(w, (2, 0, 1)).reshape(3 * C, C)
        for w in (dp_w1, dp_w2, pp_w1, pp_w2, ep_w1, ep_w2)
    ]).astype(BF16)

    vecs = jnp.stack([
        dp_b1, dp_g1, dp_be1, dp_b2, dp_g2, dp_be2, dp_lw.reshape(T),
        pp_b1, pp_g1, pp_be1, pp_b2, pp_g2, pp_be2, pp_lw.reshape(T),
        ep_b1, ep_g1, ep_be1, ep_b2, ep_g2, ep_be2, ep_lw.reshape(T),
        pe_w[:, 0, 0], pe_w[:, 0, 1], pe_w[:, 0, 2],
        ee_w[:, 0, 0], ee_w[:, 0, 1], ee_w[:, 0, 2],
        pe_b, ee_b,
    ]).astype(F32)

    dur = duration_target.astype(jnp.int32)

    gidx, vmask, mel = _tc0(maxlen, dur)

    dpred = pl.pallas_call(
        _tca_body,
        grid=(B // NB,),
        in_specs=[pl.BlockSpec(memory_space=pltpu.SMEM),
                  pl.BlockSpec((NB, C, T), lambda s: (s, 0, 0)),
                  _full((6, 3 * C, C)), _full((29, T))],
        out_specs=pl.BlockSpec((1, NB, C), lambda s: (s, 0, 0)),
        out_shape=jax.ShapeDtypeStruct((B // NB, NB, C), F32),
        compiler_params=pltpu.CompilerParams(
            dimension_semantics=("parallel",)),
    )(lbs, x, w6, vecs)

    x2_flat = _sc_gather(x.reshape(NROWS, C), gidx.reshape(NROWS))

    out, ppred, epred = pl.pallas_call(
        _tcb_body,
        grid=(B // NB,),
        in_specs=[pl.BlockSpec(memory_space=pltpu.SMEM),
                  pl.BlockSpec((NB, C, T), lambda s: (s, 0, 0)),
                  pl.BlockSpec((NB, T), lambda s: (s, 0)),
                  pl.BlockSpec((NB, T), lambda s: (s, 0)),
                  pl.BlockSpec((NB, T), lambda s: (s, 0)),
                  _full((6, 3 * C, C)), _full((29, T))],
        out_specs=(pl.BlockSpec((NB, C, T), lambda s: (s, 0, 0)),
                   pl.BlockSpec((1, NB, C), lambda s: (s, 0, 0)),
                   pl.BlockSpec((1, NB, C), lambda s: (s, 0, 0))),
        out_shape=(jax.ShapeDtypeStruct((B, C, T), F32),
                   jax.ShapeDtypeStruct((B // NB, NB, C), F32),
                   jax.ShapeDtypeStruct((B // NB, NB, C), F32)),
        compiler_params=pltpu.CompilerParams(
            dimension_semantics=("parallel",)),
    )(lbs, x2_flat.reshape(B, C, T), vmask, pitch_target, energy_target,
      w6, vecs)

    return (out, mel.reshape(B), dpred.reshape(B, C), ppred.reshape(B, C),
            epred.reshape(B, C))
